# baseline (speedup 1.0000x reference)
import functools

import jax
import jax.numpy as jnp
from jax import lax
from jax.experimental import pallas as pl
from jax.experimental.pallas import tpu as pltpu
from jax.experimental.pallas import tpu_sc as plsc

NC = 2
NS = 16
L = 16
CHUNK = 128
FAST_SHARE = 0.5
PACK_B = 14
PACK_M = (1 << PACK_B) - 1


def _mesh():
    return plsc.VectorSubcoreMesh(
        core_axis_name="c", subcore_axis_name="s",
        num_cores=NC, num_subcores=NS)


def _make_deg_kernel(n_pad, kc0, kc1):
    rows_per_tile = n_pad // NS
    zsz = ((rows_per_tile + L - 1) // L) * L
    kc_max = max(kc0, kc1)

    @functools.partial(
        pl.kernel,
        out_type=jax.ShapeDtypeStruct((NC, n_pad), jnp.float32),
        mesh=_mesh(),
        scratch_types=[
            pltpu.VMEM((kc_max, CHUNK), jnp.int32),
            pltpu.VMEM((CHUNK,), jnp.float32),
            pltpu.VMEM((zsz,), jnp.float32),
            pltpu.VMEM_SHARED((n_pad,), jnp.float32),
        ],
    )
    def deg_kernel(dst_hbm, out_hbm, idx_v, ones_v, zbuf, deg_sh):
        c = lax.axis_index("c")
        s = lax.axis_index("s")
        wid = c * NS + s
        kc_c = jnp.where(c == 0, kc0, kc1)
        for i in range(CHUNK // L):
            ones_v[pl.ds(i * L, L)] = jnp.ones((L,), jnp.float32)

        def zfill(i, carry):
            zbuf[pl.ds(i * L, L)] = jnp.zeros((L,), jnp.float32)
            return carry

        lax.fori_loop(0, zsz // L, zfill, 0)
        pltpu.sync_copy(zbuf.at[pl.ds(0, rows_per_tile)],
                        deg_sh.at[pl.ds(s * rows_per_tile, rows_per_tile)])
        pltpu.sync_copy(dst_hbm.at[wid], idx_v)
        plsc.subcore_barrier()

        def body(j, carry):
            pltpu.sync_copy(ones_v, deg_sh.at[idx_v.at[j]], add=True)
            return carry

        lax.fori_loop(0, kc_c, body, 0)
        plsc.subcore_barrier()
        pltpu.sync_copy(
            deg_sh.at[pl.ds(s * rows_per_tile, rows_per_tile)],
            out_hbm.at[c, pl.ds(s * rows_per_tile, rows_per_tile)])

    return deg_kernel


def _make_edge_kernel(n_pad, kc0, kc1, d):
    rows_per_tile = n_pad // NS
    kc_max = max(kc0, kc1)

    @functools.partial(
        pl.kernel,
        out_type=jax.ShapeDtypeStruct((NC, n_pad, d), jnp.float32),
        mesh=_mesh(),
        scratch_types=[
            pltpu.VMEM((kc_max, CHUNK), jnp.int32),
            pltpu.VMEM((2, CHUNK), jnp.int32),
            pltpu.VMEM((2, CHUNK), jnp.int32),
            pltpu.VMEM((CHUNK, d), jnp.float32),
            pltpu.VMEM((CHUNK, d), jnp.float32),
            pltpu.VMEM_SHARED((n_pad, d), jnp.float32),
            pltpu.SemaphoreType.DMA,
            pltpu.SemaphoreType.DMA,
            pltpu.SemaphoreType.DMA,
            pltpu.SemaphoreType.DMA,
        ],
    )
    def edge_kernel(y_hbm, pk_hbm, out_hbm,
                    pk_v, ua, ub, rows_a, rows_b, s_sh,
                    sem_ga, sem_gb, sem_sa, sem_sb):
        c = lax.axis_index("c")
        s = lax.axis_index("s")
        wid = c * NS + s
        kc_c = jnp.where(c == 0, kc0, kc1)

        def zfill(i, carry):
            r = i // (d // L)
            k = (i % (d // L)) * L
            rows_a[r, pl.ds(k, L)] = jnp.zeros((L,), jnp.float32)
            return carry

        lax.fori_loop(0, CHUNK * (d // L), zfill, 0)
        for t in range(rows_per_tile // CHUNK):
            pltpu.sync_copy(
                rows_a,
                s_sh.at[pl.ds(s * rows_per_tile + t * CHUNK, CHUNK)])
        pltpu.sync_copy(pk_hbm.at[wid], pk_v)
        plsc.subcore_barrier()

        def unpack(j, u):
            for i in range(CHUNK // L):
                v = pk_v[j, pl.ds(i * L, L)]
                u[0, pl.ds(i * L, L)] = jnp.bitwise_and(v, PACK_M)
                u[1, pl.ds(i * L, L)] = jnp.right_shift(v, PACK_B)

        unpack(0, ua)
        pltpu.async_copy(y_hbm.at[ua.at[0]], rows_a, sem_ga)

        def body(p, carry):
            j0 = 2 * p
            j1 = j0 + 1
            pltpu.make_async_copy(
                y_hbm.at[ua.at[0]], rows_a, sem_ga).wait()
            pltpu.async_copy(rows_a, s_sh.at[ua.at[1]], sem_sa, add=True)

            @pl.when(p > 0)
            def _():
                pltpu.make_async_copy(
                    rows_b, s_sh.at[ub.at[1]], sem_sb).wait()

            unpack(j1, ub)
            pltpu.async_copy(y_hbm.at[ub.at[0]], rows_b, sem_gb)
            pltpu.make_async_copy(
                y_hbm.at[ub.at[0]], rows_b, sem_gb).wait()
            pltpu.async_copy(rows_b, s_sh.at[ub.at[1]], sem_sb, add=True)
            pltpu.make_async_copy(
                rows_a, s_sh.at[ua.at[1]], sem_sa).wait()

            @pl.when(j0 + 2 < kc_c)
            def _():
                unpack(j0 + 2, ua)
                pltpu.async_copy(y_hbm.at[ua.at[0]], rows_a, sem_ga)

            return carry

        lax.fori_loop(0, kc_c // 2, body, 0)
        pltpu.make_async_copy(
            rows_b, s_sh.at[ub.at[1]], sem_sb).wait()
        plsc.subcore_barrier()
        for t in range(rows_per_tile // CHUNK):
            base = s * rows_per_tile + t * CHUNK
            pltpu.sync_copy(s_sh.at[pl.ds(base, CHUNK)],
                            out_hbm.at[c, pl.ds(base, CHUNK)])

    return edge_kernel


def _scale_body(degp_ref, x_ref, y_ref):
    deg = degp_ref[0] + degp_ref[1] + 1.0
    dis = lax.rsqrt(deg)
    y_ref[...] = x_ref[...] * dis


def _out_body(degp_ref, s_ref, x_ref, wl_ref, bl_ref, wh_ref, bh_ref,
              lo_ref, hi_ref):
    deg = degp_ref[0] + degp_ref[1] + 1.0
    dis = lax.rsqrt(deg)
    stot = s_ref[0] + s_ref[1]
    agg = dis * stot + (dis * dis) * x_ref[...]
    lo_ref[...] = jnp.dot(agg, wl_ref[...],
                          preferred_element_type=jnp.float32) + bl_ref[...]
    hi_ref[...] = jnp.dot(agg, wh_ref[...],
                          preferred_element_type=jnp.float32) + bh_ref[...]


def kernel(x, edge_index, W_low, b_low, W_high, b_high):
    n, d = x.shape
    hid = W_low.shape[1]
    e = edge_index.shape[1]

    n_pad = ((n + 1 + 2047) // 2048) * 2048

    src = edge_index[0].astype(jnp.int32)
    dst = edge_index[1].astype(jnp.int32)

    per_tile = (e + NS - 1) // NS
    kc0 = -(-int(per_tile * FAST_SHARE) // CHUNK)
    kc0 = kc0 + (kc0 % 2)
    cap0 = NS * kc0 * CHUNK
    kc1 = -(-(e - cap0) // (NS * CHUNK))
    kc1 = max(2, kc1 + (kc1 % 2))
    kc_max = max(kc0, kc1)
    e_pad = (kc0 + kc1) * NS * CHUNK

    def split_pad(v, fill):
        flat = jnp.concatenate([v, jnp.full((e_pad - e,), fill, jnp.int32)])
        b0 = jnp.pad(flat[:cap0].reshape(NS, kc0, CHUNK),
                     ((0, 0), (0, kc_max - kc0), (0, 0)),
                     constant_values=fill)
        b1 = jnp.pad(flat[cap0:].reshape(NS, kc1, CHUNK),
                     ((0, 0), (0, kc_max - kc1), (0, 0)),
                     constant_values=fill)
        return jnp.concatenate([b0, b1], axis=0)

    src_sp = split_pad(src, 0)
    dst_sp = split_pad(dst, n)
    pk_sp = src_sp | (dst_sp << PACK_B)

    x_pad = jnp.pad(x, ((0, n_pad - n), (0, 0)))
    degp = _make_deg_kernel(n_pad, kc0, kc1)(dst_sp)
    degp3 = degp.reshape(NC, n_pad, 1)

    blk = 512
    grid = n_pad // blk
    y = pl.pallas_call(
        _scale_body,
        grid=(grid,),
        in_specs=[
            pl.BlockSpec((NC, blk, 1), lambda i: (0, i, 0)),
            pl.BlockSpec((blk, d), lambda i: (i, 0)),
        ],
        out_specs=pl.BlockSpec((blk, d), lambda i: (i, 0)),
        out_shape=jax.ShapeDtypeStruct((n_pad, d), jnp.float32),
    )(degp3, x_pad)

    S = _make_edge_kernel(n_pad, kc0, kc1, d)(y, pk_sp)

    lo, hi = pl.pallas_call(
        _out_body,
        grid=(grid,),
        in_specs=[
            pl.BlockSpec((NC, blk, 1), lambda i: (0, i, 0)),
            pl.BlockSpec((NC, blk, d), lambda i: (0, i, 0)),
            pl.BlockSpec((blk, d), lambda i: (i, 0)),
            pl.BlockSpec((d, hid), lambda i: (0, 0)),
            pl.BlockSpec((1, hid), lambda i: (0, 0)),
            pl.BlockSpec((d, hid), lambda i: (0, 0)),
            pl.BlockSpec((1, hid), lambda i: (0, 0)),
        ],
        out_specs=[
            pl.BlockSpec((blk, hid), lambda i: (i, 0)),
            pl.BlockSpec((blk, hid), lambda i: (i, 0)),
        ],
        out_shape=[
            jax.ShapeDtypeStruct((n_pad, hid), jnp.float32),
            jax.ShapeDtypeStruct((n_pad, hid), jnp.float32),
        ],
    )(degp3, S, x_pad, W_low, b_low.reshape(1, hid),
      W_high, b_high.reshape(1, hid))

    return (lo[:n], hi[:n])


# speedup vs baseline: 1.0487x; 1.0487x over previous
import functools

import jax
import jax.numpy as jnp
from jax import lax
from jax.experimental import pallas as pl
from jax.experimental.pallas import tpu as pltpu
from jax.experimental.pallas import tpu_sc as plsc

NC = 2
NS = 16
L = 16
CHUNK = 128
FAST_SHARE = 0.68
PACK_B = 14
PACK_M = (1 << PACK_B) - 1


def _mesh():
    return plsc.VectorSubcoreMesh(
        core_axis_name="c", subcore_axis_name="s",
        num_cores=NC, num_subcores=NS)


def _make_deg_kernel(n_pad, kc0, kc1):
    rows_per_tile = n_pad // NS
    zsz = ((rows_per_tile + L - 1) // L) * L
    kc_max = max(kc0, kc1)

    @functools.partial(
        pl.kernel,
        out_type=jax.ShapeDtypeStruct((NC, n_pad), jnp.float32),
        mesh=_mesh(),
        scratch_types=[
            pltpu.VMEM((kc_max, CHUNK), jnp.int32),
            pltpu.VMEM((CHUNK,), jnp.float32),
            pltpu.VMEM((zsz,), jnp.float32),
            pltpu.VMEM_SHARED((n_pad,), jnp.float32),
        ],
    )
    def deg_kernel(dst_hbm, out_hbm, idx_v, ones_v, zbuf, deg_sh):
        c = lax.axis_index("c")
        s = lax.axis_index("s")
        wid = c * NS + s
        kc_c = jnp.where(c == 0, kc0, kc1)
        for i in range(CHUNK // L):
            ones_v[pl.ds(i * L, L)] = jnp.ones((L,), jnp.float32)

        def zfill(i, carry):
            zbuf[pl.ds(i * L, L)] = jnp.zeros((L,), jnp.float32)
            return carry

        lax.fori_loop(0, zsz // L, zfill, 0)
        pltpu.sync_copy(zbuf.at[pl.ds(0, rows_per_tile)],
                        deg_sh.at[pl.ds(s * rows_per_tile, rows_per_tile)])
        pltpu.sync_copy(dst_hbm.at[wid], idx_v)
        plsc.subcore_barrier()

        def body(j, carry):
            pltpu.sync_copy(ones_v, deg_sh.at[idx_v.at[j]], add=True)
            return carry

        lax.fori_loop(0, kc_c, body, 0)
        plsc.subcore_barrier()
        pltpu.sync_copy(
            deg_sh.at[pl.ds(s * rows_per_tile, rows_per_tile)],
            out_hbm.at[c, pl.ds(s * rows_per_tile, rows_per_tile)])

    return deg_kernel


def _make_edge_kernel(n_pad, kc0, kc1, d):
    rows_per_tile = n_pad // NS
    kc_max = max(kc0, kc1)

    @functools.partial(
        pl.kernel,
        out_type=jax.ShapeDtypeStruct((NC, n_pad, d), jnp.float32),
        mesh=_mesh(),
        scratch_types=[
            pltpu.VMEM((kc_max, CHUNK), jnp.int32),
            pltpu.VMEM((2, CHUNK), jnp.int32),
            pltpu.VMEM((2, CHUNK), jnp.int32),
            pltpu.VMEM((CHUNK, d), jnp.float32),
            pltpu.VMEM((CHUNK, d), jnp.float32),
            pltpu.VMEM_SHARED((n_pad, d), jnp.float32),
            pltpu.SemaphoreType.DMA,
            pltpu.SemaphoreType.DMA,
            pltpu.SemaphoreType.DMA,
            pltpu.SemaphoreType.DMA,
        ],
    )
    def edge_kernel(y_hbm, pk_hbm, out_hbm,
                    pk_v, ua, ub, rows_a, rows_b, s_sh,
                    sem_ga, sem_gb, sem_sa, sem_sb):
        c = lax.axis_index("c")
        s = lax.axis_index("s")
        wid = c * NS + s
        kc_c = jnp.where(c == 0, kc0, kc1)

        def zfill(i, carry):
            r = i // (d // L)
            k = (i % (d // L)) * L
            rows_a[r, pl.ds(k, L)] = jnp.zeros((L,), jnp.float32)
            return carry

        lax.fori_loop(0, CHUNK * (d // L), zfill, 0)
        for t in range(rows_per_tile // CHUNK):
            pltpu.sync_copy(
                rows_a,
                s_sh.at[pl.ds(s * rows_per_tile + t * CHUNK, CHUNK)])
        pltpu.sync_copy(pk_hbm.at[wid], pk_v)
        plsc.subcore_barrier()

        def unpack(j, u):
            for i in range(CHUNK // L):
                v = pk_v[j, pl.ds(i * L, L)]
                u[0, pl.ds(i * L, L)] = jnp.bitwise_and(v, PACK_M)
                u[1, pl.ds(i * L, L)] = jnp.right_shift(v, PACK_B)

        unpack(0, ua)
        pltpu.async_copy(y_hbm.at[ua.at[0]], rows_a, sem_ga)

        def body(p, carry):
            j0 = 2 * p
            j1 = j0 + 1
            pltpu.make_async_copy(
                y_hbm.at[ua.at[0]], rows_a, sem_ga).wait()
            pltpu.async_copy(rows_a, s_sh.at[ua.at[1]], sem_sa, add=True)

            @pl.when(p > 0)
            def _():
                pltpu.make_async_copy(
                    rows_b, s_sh.at[ub.at[1]], sem_sb).wait()

            unpack(j1, ub)
            pltpu.async_copy(y_hbm.at[ub.at[0]], rows_b, sem_gb)
            pltpu.make_async_copy(
                y_hbm.at[ub.at[0]], rows_b, sem_gb).wait()
            pltpu.async_copy(rows_b, s_sh.at[ub.at[1]], sem_sb, add=True)
            pltpu.make_async_copy(
                rows_a, s_sh.at[ua.at[1]], sem_sa).wait()

            @pl.when(j0 + 2 < kc_c)
            def _():
                unpack(j0 + 2, ua)
                pltpu.async_copy(y_hbm.at[ua.at[0]], rows_a, sem_ga)

            return carry

        lax.fori_loop(0, kc_c // 2, body, 0)
        pltpu.make_async_copy(
            rows_b, s_sh.at[ub.at[1]], sem_sb).wait()
        plsc.subcore_barrier()
        for t in range(rows_per_tile // CHUNK):
            base = s * rows_per_tile + t * CHUNK
            pltpu.sync_copy(s_sh.at[pl.ds(base, CHUNK)],
                            out_hbm.at[c, pl.ds(base, CHUNK)])

    return edge_kernel


def _scale_body(degp_ref, x_ref, y_ref):
    deg = degp_ref[0] + degp_ref[1] + 1.0
    dis = lax.rsqrt(deg)
    y_ref[...] = x_ref[...] * dis


def _out_body(degp_ref, s_ref, x_ref, wl_ref, bl_ref, wh_ref, bh_ref,
              lo_ref, hi_ref):
    deg = degp_ref[0] + degp_ref[1] + 1.0
    dis = lax.rsqrt(deg)
    stot = s_ref[0] + s_ref[1]
    agg = dis * stot + (dis * dis) * x_ref[...]
    lo_ref[...] = jnp.dot(agg, wl_ref[...],
                          preferred_element_type=jnp.float32) + bl_ref[...]
    hi_ref[...] = jnp.dot(agg, wh_ref[...],
                          preferred_element_type=jnp.float32) + bh_ref[...]


def kernel(x, edge_index, W_low, b_low, W_high, b_high):
    n, d = x.shape
    hid = W_low.shape[1]
    e = edge_index.shape[1]

    n_pad = ((n + 1 + 2047) // 2048) * 2048

    src = edge_index[0].astype(jnp.int32)
    dst = edge_index[1].astype(jnp.int32)

    per_tile = (e + NS - 1) // NS
    kc0 = -(-int(per_tile * FAST_SHARE) // CHUNK)
    kc0 = kc0 + (kc0 % 2)
    cap0 = NS * kc0 * CHUNK
    kc1 = -(-(e - cap0) // (NS * CHUNK))
    kc1 = max(2, kc1 + (kc1 % 2))
    kc_max = max(kc0, kc1)
    e_pad = (kc0 + kc1) * NS * CHUNK

    def split_pad(v, fill):
        flat = jnp.concatenate([v, jnp.full((e_pad - e,), fill, jnp.int32)])
        b0 = jnp.pad(flat[:cap0].reshape(NS, kc0, CHUNK),
                     ((0, 0), (0, kc_max - kc0), (0, 0)),
                     constant_values=fill)
        b1 = jnp.pad(flat[cap0:].reshape(NS, kc1, CHUNK),
                     ((0, 0), (0, kc_max - kc1), (0, 0)),
                     constant_values=fill)
        return jnp.concatenate([b0, b1], axis=0)

    src_sp = split_pad(src, 0)
    dst_sp = split_pad(dst, n)
    pk_sp = src_sp | (dst_sp << PACK_B)

    x_pad = jnp.pad(x, ((0, n_pad - n), (0, 0)))
    degp = _make_deg_kernel(n_pad, kc0, kc1)(dst_sp)
    degp3 = degp.reshape(NC, n_pad, 1)

    blk = 512
    grid = n_pad // blk
    y = pl.pallas_call(
        _scale_body,
        grid=(grid,),
        in_specs=[
            pl.BlockSpec((NC, blk, 1), lambda i: (0, i, 0)),
            pl.BlockSpec((blk, d), lambda i: (i, 0)),
        ],
        out_specs=pl.BlockSpec((blk, d), lambda i: (i, 0)),
        out_shape=jax.ShapeDtypeStruct((n_pad, d), jnp.float32),
    )(degp3, x_pad)

    S = _make_edge_kernel(n_pad, kc0, kc1, d)(y, pk_sp)

    lo, hi = pl.pallas_call(
        _out_body,
        grid=(grid,),
        in_specs=[
            pl.BlockSpec((NC, blk, 1), lambda i: (0, i, 0)),
            pl.BlockSpec((NC, blk, d), lambda i: (0, i, 0)),
            pl.BlockSpec((blk, d), lambda i: (i, 0)),
            pl.BlockSpec((d, hid), lambda i: (0, 0)),
            pl.BlockSpec((1, hid), lambda i: (0, 0)),
            pl.BlockSpec((d, hid), lambda i: (0, 0)),
            pl.BlockSpec((1, hid), lambda i: (0, 0)),
        ],
        out_specs=[
            pl.BlockSpec((blk, hid), lambda i: (i, 0)),
            pl.BlockSpec((blk, hid), lambda i: (i, 0)),
        ],
        out_shape=[
            jax.ShapeDtypeStruct((n_pad, hid), jnp.float32),
            jax.ShapeDtypeStruct((n_pad, hid), jnp.float32),
        ],
    )(degp3, S, x_pad, W_low, b_low.reshape(1, hid),
      W_high, b_high.reshape(1, hid))

    return (lo[:n], hi[:n])
